# baseline (speedup 1.0000x reference)
import functools
import math

import jax
import jax.numpy as jnp
from jax import lax
from jax.experimental import pallas as pl
from jax.experimental.pallas import tpu as pltpu
from jax.experimental.pallas import tpu_sc as plsc


def _corr_kernel(q_ref, k_ref, out_ref, *, blk_r, l_len):
  ib = pl.program_id(1)
  jrev = jnp.where(
      lax.broadcasted_iota(jnp.int32, (blk_r, blk_r), 0)
      + lax.broadcasted_iota(jnp.int32, (blk_r, blk_r), 1) == blk_r - 1,
      jnp.float32(1.0), jnp.float32(0.0))
  qb = lax.dot_general(
      q_ref[0], jrev, (((1,), (0,)), ((), ())),
      preferred_element_type=jnp.float32)
  m = lax.dot_general(
      qb, k_ref[0], (((0,), (0,)), ((), ())),
      preferred_element_type=jnp.float32)
  slab = 128
  rowsum = jnp.zeros((1, l_len), jnp.float32)
  for s0 in range(0, blk_r, slab):
    msl = m[s0:s0 + slab] if s0 == 0 else pltpu.roll(m[s0:s0 + slab], s0, axis=1)
    aligned = pltpu.roll(msl, 0, axis=1, stride=1, stride_axis=0)
    rowsum = rowsum + jnp.sum(aligned, axis=0, keepdims=True)
  contrib = pltpu.roll(
      rowsum, (2 * l_len - ib * blk_r - blk_r + 1) % l_len, axis=1)

  @pl.when(ib == 0)
  def _init():
    out_ref[0] = jnp.zeros_like(contrib)

  out_ref[0] += contrib


def _mean_corr(q3, k3, blk_r):
  b_sz, f_len, l_len = q3.shape
  grid = (b_sz, l_len // blk_r)
  out = pl.pallas_call(
      functools.partial(_corr_kernel, blk_r=blk_r, l_len=l_len),
      grid=grid,
      in_specs=[
          pl.BlockSpec((1, f_len, blk_r), lambda b, i: (b, 0, i)),
          pl.BlockSpec((1, f_len, l_len), lambda b, i: (b, 0, 0)),
      ],
      out_specs=pl.BlockSpec((1, 1, l_len), lambda b, i: (b, 0, 0)),
      out_shape=jax.ShapeDtypeStruct((b_sz, 1, l_len), jnp.float32),
      compiler_params=pltpu.CompilerParams(
          dimension_semantics=("parallel", "arbitrary"),
      ),
  )(q3, k3)
  return out


def _make_topk_sc(b_sz, l_len, top_k, inv_f):
  lanes = 16
  n_chunks = l_len // lanes
  neg_inf = jnp.float32(-3.0e38)
  mesh = plsc.VectorSubcoreMesh(core_axis_name="c", subcore_axis_name="s")

  @functools.partial(
      pl.kernel,
      mesh=mesh,
      out_type=(
          jax.ShapeDtypeStruct((lanes,), jnp.int32),
          jax.ShapeDtypeStruct((b_sz, lanes), jnp.float32),
      ),
      scratch_types=[
          pltpu.VMEM((b_sz, l_len), jnp.float32),
          pltpu.VMEM((b_sz, l_len), jnp.float32),
          pltpu.VMEM((l_len,), jnp.float32),
          pltpu.VMEM((lanes,), jnp.int32),
          pltpu.VMEM((b_sz, lanes), jnp.float32),
      ],
      compiler_params=pltpu.CompilerParams(needs_layout_passes=False),
  )
  def topk_kernel(corr_hbm, idx_hbm, w_hbm, d_v, mv_v, bm_v, idx_v, w_v):
    cid = lax.axis_index("c")
    sid = lax.axis_index("s")
    is_leader = jnp.logical_and(cid == 0, sid == 0)

    @pl.when(is_leader)
    def _body():
      pltpu.sync_copy(corr_hbm, d_v)
      lane_iota = lax.iota(jnp.int32, lanes)
      scale = jnp.float32(inv_f)

      def scale_chunk(c, carry):
        sl = pl.ds(c * lanes, lanes)
        t = jnp.int32(l_len) - c * lanes - lane_iota
        u = jnp.where(t >= l_len, t - l_len, t)
        acc = jnp.zeros((lanes,), jnp.float32)
        for b in range(b_sz):
          v = plsc.load_gather(
              d_v, [jnp.full((lanes,), b, jnp.int32), u]) * scale
          mv_v[b, sl] = v
          acc = acc + v
        bm_v[sl] = acc * jnp.float32(1.0 / b_sz)
        return carry

      lax.fori_loop(0, n_chunks, scale_chunk, 0)

      found = []
      for _ in range(top_k):
        def scan_chunk(c, carry):
          run_max, run_idx = carry
          v = bm_v[pl.ds(c * lanes, lanes)]
          better = v > run_max
          run_max = jnp.where(better, v, run_max)
          run_idx = jnp.where(better, c * lanes + lane_iota, run_idx)
          return run_max, run_idx

        run_max, run_idx = lax.fori_loop(
            0, n_chunks, scan_chunk,
            (jnp.full((lanes,), neg_inf, jnp.float32),
             jnp.zeros((lanes,), jnp.int32)))
        skeys, _ = plsc.sort_key_val(run_max, run_idx, descending=True)
        gmax = skeys[0]
        cand = jnp.where(run_max == gmax, run_idx, jnp.int32(2**31 - 1))
        scand, _ = plsc.sort_key_val(cand, run_idx, descending=False)
        sel = scand[0]
        found.append(sel)
        c_sel = sel // lanes
        sl = pl.ds(c_sel * lanes, lanes)
        bm_v[sl] = jnp.where(lane_iota == sel % lanes, neg_inf, bm_v[sl])

      idxs = jnp.zeros((lanes,), jnp.int32)
      for i, sel in enumerate(found):
        idxs = jnp.where(lane_iota == i, sel, idxs)
      idx_v[...] = idxs

      valid = lane_iota < top_k
      gather_idx = jnp.where(valid, idxs, 0)
      for b in range(b_sz):
        logits = plsc.load_gather(
            mv_v, [jnp.full((lanes,), b, jnp.int32), gather_idx])
        logits = jnp.where(valid, logits, neg_inf)
        skeys, _ = plsc.sort_key_val(logits, gather_idx, descending=True)
        mx = skeys[0]
        e = jnp.where(valid, jnp.exp(logits - mx), jnp.float32(0.0))
        z = jnp.float32(0.0)
        for i in range(lanes):
          z = z + e[i]
        w_v[b, :] = jnp.where(lane_iota == top_k, z, e)

      pltpu.sync_copy(idx_v, idx_hbm)
      pltpu.sync_copy(w_v, w_hbm)

  return topk_kernel


def _agg_kernel(idx_ref, w_ref, v_ref, out_ref, *, top_k):
  b = pl.program_id(0)
  v = v_ref[0]
  l_len = v.shape[-1]
  acc = jnp.zeros_like(v)
  inv_z = 1.0 / w_ref[b * 16 + top_k]
  for i in range(top_k):
    shift = (l_len - idx_ref[i]) & (l_len - 1)
    acc = acc + pltpu.roll(v, shift, axis=2) * (w_ref[b * 16 + i] * inv_z)
  out_ref[0] = acc


def _aggregate(values, idx, w, top_k):
  b_sz, h_sz, d_sz, l_len = values.shape
  h_blk = 1
  grid_spec = pltpu.PrefetchScalarGridSpec(
      num_scalar_prefetch=2,
      grid=(b_sz, h_sz // h_blk),
      in_specs=[
          pl.BlockSpec((1, h_blk, d_sz, l_len), lambda b, h, i_r, w_r: (b, h, 0, 0)),
      ],
      out_specs=pl.BlockSpec((1, h_blk, d_sz, l_len), lambda b, h, i_r, w_r: (b, h, 0, 0)),
  )
  return pl.pallas_call(
      functools.partial(_agg_kernel, top_k=top_k),
      grid_spec=grid_spec,
      out_shape=jax.ShapeDtypeStruct(values.shape, values.dtype),
      compiler_params=pltpu.CompilerParams(
          dimension_semantics=("parallel", "parallel"),
      ),
  )(idx, w.reshape(-1), values)


def kernel(queries, keys, values, attn_mask):
  del attn_mask
  b_sz, h_sz, l_len, d_sz = queries.shape
  f_len = h_sz * d_sz
  top_k = int(math.log(l_len))

  q3 = queries.transpose(0, 1, 3, 2).reshape(b_sz, f_len, l_len)
  k3 = keys.transpose(0, 1, 3, 2).reshape(b_sz, f_len, l_len)

  corr_sum = _mean_corr(q3, k3, blk_r=1024)
  corr_sum = corr_sum.reshape(b_sz, l_len)

  topk_fn = _make_topk_sc(b_sz, l_len, top_k, 1.0 / f_len)
  idx, w = topk_fn(corr_sum)

  vp = values.transpose(0, 1, 3, 2)
  outp = _aggregate(vp, idx, w, top_k)
  return outp.transpose(0, 1, 3, 2)


# speedup vs baseline: 1.0381x; 1.0381x over previous
import functools
import math

import jax
import jax.numpy as jnp
from jax import lax
from jax.experimental import pallas as pl
from jax.experimental.pallas import tpu as pltpu
from jax.experimental.pallas import tpu_sc as plsc


def _corr_kernel(q_ref, k_ref, out_ref, *, blk_r, l_len):
  ib = pl.program_id(1)
  jrev = jnp.where(
      lax.broadcasted_iota(jnp.int32, (blk_r, blk_r), 0)
      + lax.broadcasted_iota(jnp.int32, (blk_r, blk_r), 1) == blk_r - 1,
      jnp.float32(1.0), jnp.float32(0.0))
  qb = lax.dot_general(
      q_ref[0], jrev, (((1,), (0,)), ((), ())),
      preferred_element_type=jnp.float32)
  m = lax.dot_general(
      qb, k_ref[0], (((0,), (0,)), ((), ())),
      preferred_element_type=jnp.float32)
  slab = 128
  rowsum = jnp.zeros((1, l_len), jnp.float32)
  for s0 in range(0, blk_r, slab):
    msl = m[s0:s0 + slab] if s0 == 0 else pltpu.roll(m[s0:s0 + slab], s0, axis=1)
    aligned = pltpu.roll(msl, 0, axis=1, stride=1, stride_axis=0)
    rowsum = rowsum + jnp.sum(aligned, axis=0, keepdims=True)
  contrib = pltpu.roll(
      rowsum, (2 * l_len - ib * blk_r - blk_r + 1) % l_len, axis=1)

  @pl.when(ib == 0)
  def _init():
    out_ref[0] = jnp.zeros_like(contrib)

  out_ref[0] += contrib


def _mean_corr(q3, k3, blk_r):
  b_sz, f_len, l_len = q3.shape
  grid = (b_sz, l_len // blk_r)
  out = pl.pallas_call(
      functools.partial(_corr_kernel, blk_r=blk_r, l_len=l_len),
      grid=grid,
      in_specs=[
          pl.BlockSpec((1, f_len, blk_r), lambda b, i: (b, 0, i)),
          pl.BlockSpec((1, f_len, l_len), lambda b, i: (b, 0, 0)),
      ],
      out_specs=pl.BlockSpec((1, 1, l_len), lambda b, i: (b, 0, 0)),
      out_shape=jax.ShapeDtypeStruct((b_sz, 1, l_len), jnp.float32),
      compiler_params=pltpu.CompilerParams(
          dimension_semantics=("parallel", "arbitrary"),
      ),
  )(q3, k3)
  return out


def _make_topk_sc(b_sz, l_len, top_k, inv_f):
  lanes = 16
  n_chunks = l_len // lanes
  neg_inf = jnp.float32(-3.0e38)
  mesh = plsc.VectorSubcoreMesh(core_axis_name="c", subcore_axis_name="s")

  @functools.partial(
      pl.kernel,
      mesh=mesh,
      out_type=(
          jax.ShapeDtypeStruct((lanes,), jnp.int32),
          jax.ShapeDtypeStruct((b_sz, lanes), jnp.float32),
      ),
      scratch_types=[
          pltpu.VMEM((b_sz, l_len), jnp.float32),
          pltpu.VMEM((b_sz, l_len), jnp.float32),
          pltpu.VMEM((l_len,), jnp.float32),
          pltpu.VMEM((l_len // 4,), jnp.float32),
          pltpu.VMEM((l_len // 4,), jnp.int32),
          pltpu.VMEM((lanes,), jnp.int32),
          pltpu.VMEM((b_sz, lanes), jnp.float32),
      ],
      compiler_params=pltpu.CompilerParams(needs_layout_passes=False),
  )
  def topk_kernel(corr_hbm, idx_hbm, w_hbm, d_v, mv_v, bm_v, bm4_v, bi4_v,
                  idx_v, w_v):
    cid = lax.axis_index("c")
    sid = lax.axis_index("s")
    is_leader = jnp.logical_and(cid == 0, sid == 0)

    @pl.when(is_leader)
    def _body():
      pltpu.sync_copy(corr_hbm, d_v)
      lane_iota = lax.iota(jnp.int32, lanes)
      scale = jnp.float32(inv_f)

      def scale_chunk(c, carry):
        sl = pl.ds(c * lanes, lanes)
        t = jnp.int32(l_len) - c * lanes - lane_iota
        u = jnp.where(t >= l_len, t - l_len, t)
        acc = jnp.zeros((lanes,), jnp.float32)
        for b in range(b_sz):
          v = plsc.load_gather(
              d_v, [jnp.full((lanes,), b, jnp.int32), u]) * scale
          mv_v[b, sl] = v
          acc = acc + v
        bm_v[sl] = acc * jnp.float32(1.0 / b_sz)
        return carry

      lax.fori_loop(0, n_chunks, scale_chunk, 0)

      def reduce4(g):
        base = g * 4 * lanes
        m = bm_v[pl.ds(base, lanes)]
        j = base + lane_iota
        for t in range(1, 4):
          vt = bm_v[pl.ds(base + t * lanes, lanes)]
          bt = vt > m
          j = jnp.where(bt, base + t * lanes + lane_iota, j)
          m = jnp.where(bt, vt, m)
        bm4_v[pl.ds(g * lanes, lanes)] = m
        bi4_v[pl.ds(g * lanes, lanes)] = j

      def red_loop(g, carry):
        reduce4(g)
        return carry

      lax.fori_loop(0, n_chunks // 4, red_loop, 0)

      found = []
      for _ in range(top_k):
        def scan_chunk(c, carry):
          run_max, run_idx = carry
          v = bm4_v[pl.ds(c * lanes, lanes)]
          ji = bi4_v[pl.ds(c * lanes, lanes)]
          better = v > run_max
          run_max = jnp.where(better, v, run_max)
          run_idx = jnp.where(better, ji, run_idx)
          return run_max, run_idx

        run_max, run_idx = lax.fori_loop(
            0, n_chunks // 4, scan_chunk,
            (jnp.full((lanes,), neg_inf, jnp.float32),
             jnp.zeros((lanes,), jnp.int32)))
        skeys, _ = plsc.sort_key_val(run_max, run_idx, descending=True)
        gmax = skeys[0]
        cand = jnp.where(run_max == gmax, run_idx, jnp.int32(2**31 - 1))
        scand, _ = plsc.sort_key_val(cand, run_idx, descending=False)
        sel = scand[0]
        found.append(sel)
        c_sel = sel // lanes
        sl = pl.ds(c_sel * lanes, lanes)
        bm_v[sl] = jnp.where(lane_iota == sel % lanes, neg_inf, bm_v[sl])
        reduce4(sel // (4 * lanes))

      idxs = jnp.zeros((lanes,), jnp.int32)
      for i, sel in enumerate(found):
        idxs = jnp.where(lane_iota == i, sel, idxs)
      idx_v[...] = idxs

      valid = lane_iota < top_k
      gather_idx = jnp.where(valid, idxs, 0)
      for b in range(b_sz):
        logits = plsc.load_gather(
            mv_v, [jnp.full((lanes,), b, jnp.int32), gather_idx])
        logits = jnp.where(valid, logits, neg_inf)
        skeys, _ = plsc.sort_key_val(logits, gather_idx, descending=True)
        mx = skeys[0]
        e = jnp.where(valid, jnp.exp(logits - mx), jnp.float32(0.0))
        z = jnp.float32(0.0)
        for i in range(lanes):
          z = z + e[i]
        w_v[b, :] = jnp.where(lane_iota == top_k, z, e)

      pltpu.sync_copy(idx_v, idx_hbm)
      pltpu.sync_copy(w_v, w_hbm)

  return topk_kernel


def _agg_kernel(idx_ref, w_ref, v_ref, out_ref, *, top_k):
  b = pl.program_id(0)
  v = v_ref[0]
  l_len = v.shape[-1]
  acc = jnp.zeros_like(v)
  inv_z = 1.0 / w_ref[b * 16 + top_k]
  for i in range(top_k):
    shift = (l_len - idx_ref[i]) & (l_len - 1)
    acc = acc + pltpu.roll(v, shift, axis=2) * (w_ref[b * 16 + i] * inv_z)
  out_ref[0] = acc


def _aggregate(values, idx, w, top_k):
  b_sz, h_sz, d_sz, l_len = values.shape
  h_blk = 1
  grid_spec = pltpu.PrefetchScalarGridSpec(
      num_scalar_prefetch=2,
      grid=(b_sz, h_sz // h_blk),
      in_specs=[
          pl.BlockSpec((1, h_blk, d_sz, l_len), lambda b, h, i_r, w_r: (b, h, 0, 0)),
      ],
      out_specs=pl.BlockSpec((1, h_blk, d_sz, l_len), lambda b, h, i_r, w_r: (b, h, 0, 0)),
  )
  return pl.pallas_call(
      functools.partial(_agg_kernel, top_k=top_k),
      grid_spec=grid_spec,
      out_shape=jax.ShapeDtypeStruct(values.shape, values.dtype),
      compiler_params=pltpu.CompilerParams(
          dimension_semantics=("parallel", "parallel"),
      ),
  )(idx, w.reshape(-1), values)


def kernel(queries, keys, values, attn_mask):
  del attn_mask
  b_sz, h_sz, l_len, d_sz = queries.shape
  f_len = h_sz * d_sz
  top_k = int(math.log(l_len))

  q3 = queries.transpose(0, 1, 3, 2).reshape(b_sz, f_len, l_len)
  k3 = keys.transpose(0, 1, 3, 2).reshape(b_sz, f_len, l_len)

  corr_sum = _mean_corr(q3, k3, blk_r=512)
  corr_sum = corr_sum.reshape(b_sz, l_len)

  topk_fn = _make_topk_sc(b_sz, l_len, top_k, 1.0 / f_len)
  idx, w = topk_fn(corr_sum)

  vp = values.transpose(0, 1, 3, 2)
  outp = _aggregate(vp, idx, w, top_k)
  return outp.transpose(0, 1, 3, 2)
